# baseline (speedup 1.0000x reference)
import functools

import jax
import jax.numpy as jnp
from jax import lax
from jax.experimental import pallas as pl
from jax.experimental.pallas import tpu as pltpu
from jax.experimental.pallas import tpu_sc as plsc

_VOCAB = 1000000
_EMB = 64
_B = 16384
_L = 50

_info = plsc.get_sparse_core_info()
_NC = _info.num_cores
_NS = _info.num_subcores
_NW = _NC * _NS
_BPW = _B // _NW
_CHUNK = 128
_NCHUNK = _BPW // _CHUNK
_NXFER = _L * _NCHUNK
_DEPTH = 16


def _fire(t, w, idx_v, acc_v, sem):
    k = t % _L
    c = t // _L
    pltpu.async_copy(
        w.at[idx_v.at[k, c]],
        acc_v.at[pl.ds(c * _CHUNK, _CHUNK)],
        sem,
        add=True,
    )


def _drain(w, idx_v, acc_v, sem):
    pltpu.make_async_copy(
        w.at[idx_v.at[0, 0]],
        acc_v.at[pl.ds(0, _CHUNK)],
        sem,
    ).wait()


_HPW = _BPW // 2
_HCHUNK = _HPW // _CHUNK
_HXFER = _L * _HCHUNK


def _emb_body(seg_hbm, w_hbm, out_hbm, idx_raw, idx_v, acc_v, sem):
    wid = lax.axis_index("s") * _NC + lax.axis_index("c")

    lanes = lax.iota(jnp.int32, 16)
    row_vecs = [jnp.int32(j * 16) + lanes for j in range(_HPW // 16)]
    zeros = jnp.zeros((16,), jnp.float32)

    for p in range(2):
        base = wid * _BPW + p * _HPW

        pltpu.sync_copy(seg_hbm.at[pl.ds(base, _HPW)], idx_raw)

        def tbody(k, _):
            col = jnp.full((16,), 0, jnp.int32) + k
            for j in range(_HPW // 16):
                v = plsc.load_gather(idx_raw, [row_vecs[j], col])
                idx_v[k, j // 8, pl.ds((j % 8) * 16, 16)] = v
            return 0

        lax.fori_loop(0, _L, tbody, 0)

        def zbody(i, _):
            for j in range(128 // 16):
                acc_v[i, pl.ds(j * 16, 16)] = zeros
            return 0

        lax.fori_loop(0, _HPW, zbody, 0)

        for t in range(_DEPTH):
            _fire(t, w_hbm, idx_v, acc_v, sem)

        def pbody(t, _):
            _drain(w_hbm, idx_v, acc_v, sem)
            _fire(t + _DEPTH, w_hbm, idx_v, acc_v, sem)
            return 0

        lax.fori_loop(0, _HXFER - _DEPTH, pbody, 0)

        for _ in range(_DEPTH):
            _drain(w_hbm, idx_v, acc_v, sem)

        pltpu.sync_copy(
            acc_v.at[:, pl.ds(0, _EMB)], out_hbm.at[pl.ds(base, _HPW)]
        )


_emb_kernel = functools.partial(
    pl.kernel,
    out_type=jax.ShapeDtypeStruct((_B, _EMB), jnp.float32),
    mesh=plsc.VectorSubcoreMesh(core_axis_name="c", subcore_axis_name="s"),
    scratch_types=[
        pltpu.VMEM((_BPW // 2, 128), jnp.int32),
        pltpu.VMEM((_L, _NCHUNK // 2, _CHUNK), jnp.int32),
        pltpu.VMEM((_BPW // 2, 128), jnp.float32),
        pltpu.SemaphoreType.DMA,
    ],
    compiler_params=pltpu.CompilerParams(
        use_tc_tiling_on_sc=False, needs_layout_passes=False
    ),
)(_emb_body)


def kernel(seg_ids, weight):
    w128 = jnp.pad(weight * jnp.float32(1.0 / _L), ((0, 0), (0, 128 - _EMB)))
    seg_pad = jnp.pad(seg_ids, ((0, 0), (0, 128 - _L)))
    return _emb_kernel(seg_pad, w128)


# speedup vs baseline: 1.4287x; 1.4287x over previous
import functools

import jax
import jax.numpy as jnp
from jax import lax
from jax.experimental import pallas as pl
from jax.experimental.pallas import tpu as pltpu
from jax.experimental.pallas import tpu_sc as plsc

_VOCAB = 1000000
_EMB = 64
_B = 16384
_L = 50

_info = plsc.get_sparse_core_info()
_NC = _info.num_cores
_NS = _info.num_subcores
_NW = _NC * _NS
_BPW = _B // _NW
_CHUNK = 128
_NCHUNK = _BPW // _CHUNK
_NXFER = _L * _NCHUNK
_DEPTH = 16


def _fire(t, w, idx_v, acc_v, sem):
    k = t % _L
    c = t // _L
    pltpu.async_copy(
        w.at[idx_v.at[k, c]],
        acc_v.at[pl.ds(c * _CHUNK, _CHUNK)],
        sem,
        add=True,
    )


def _drain(w, idx_v, acc_v, sem):
    pltpu.make_async_copy(
        w.at[idx_v.at[0, 0]],
        acc_v.at[pl.ds(0, _CHUNK)],
        sem,
    ).wait()


_HPW = _BPW // 2
_HCHUNK = _HPW // _CHUNK
_HXFER = _L * _HCHUNK


def _emb_body(seg_hbm, w_hbm, out_hbm, idx_raw, idx_v, acc_v, sem):
    wid = lax.axis_index("s") * _NC + lax.axis_index("c")

    lanes = lax.iota(jnp.int32, 16)
    row_vecs = [jnp.int32(j * 16) + lanes for j in range(_HPW // 16)]
    zeros = jnp.zeros((16,), jnp.float32)

    for p in range(2):
        base = wid * _BPW + p * _HPW

        pltpu.sync_copy(seg_hbm.at[pl.ds(base, _HPW)], idx_raw)

        def tbody(k, _):
            col = jnp.full((16,), 0, jnp.int32) + k
            for j in range(_HPW // 16):
                v = plsc.load_gather(idx_raw, [row_vecs[j], col])
                idx_v[k, j // 8, pl.ds((j % 8) * 16, 16)] = v
            return 0

        lax.fori_loop(0, _L, tbody, 0)

        def zbody(i, _):
            for j in range(128 // 16):
                acc_v[i, pl.ds(j * 16, 16)] = zeros
            return 0

        lax.fori_loop(0, _HPW, zbody, 0)

        for t in range(_DEPTH):
            _fire(t, w_hbm, idx_v, acc_v, sem)

        def pbody(t, _):
            _drain(w_hbm, idx_v, acc_v, sem)
            _fire(t + _DEPTH, w_hbm, idx_v, acc_v, sem)
            return 0

        lax.fori_loop(0, _HXFER - _DEPTH, pbody, 0)

        for _ in range(_DEPTH):
            _drain(w_hbm, idx_v, acc_v, sem)

        inv = jnp.float32(1.0 / _L)

        def sbody(i, _):
            for j in range(_EMB // 16):
                sl = pl.ds(j * 16, 16)
                acc_v[i, sl] = acc_v[i, sl] * inv
            return 0

        lax.fori_loop(0, _HPW, sbody, 0)

        pltpu.sync_copy(
            acc_v.at[:, pl.ds(0, _EMB)], out_hbm.at[pl.ds(base, _HPW)]
        )


_emb_kernel = functools.partial(
    pl.kernel,
    out_type=jax.ShapeDtypeStruct((_B, _EMB), jnp.float32),
    mesh=plsc.VectorSubcoreMesh(core_axis_name="c", subcore_axis_name="s"),
    scratch_types=[
        pltpu.VMEM((_BPW // 2, 128), jnp.int32),
        pltpu.VMEM((_L, _NCHUNK // 2, _CHUNK), jnp.int32),
        pltpu.VMEM((_BPW // 2, 128), jnp.float32),
        pltpu.SemaphoreType.DMA,
    ],
    compiler_params=pltpu.CompilerParams(
        use_tc_tiling_on_sc=False, needs_layout_passes=False
    ),
)(_emb_body)


def kernel(seg_ids, weight):
    w128 = jnp.pad(weight, ((0, 0), (0, 128 - _EMB)))
    seg_pad = jnp.pad(seg_ids, ((0, 0), (0, 128 - _L)))
    return _emb_kernel(seg_pad, w128)


# speedup vs baseline: 1.4955x; 1.0467x over previous
import functools

import jax
import jax.numpy as jnp
from jax import lax
from jax.experimental import pallas as pl
from jax.experimental.pallas import tpu as pltpu
from jax.experimental.pallas import tpu_sc as plsc

_VOCAB = 1000000
_EMB = 64
_B = 16384
_L = 50

_info = plsc.get_sparse_core_info()
_NC = _info.num_cores
_NS = _info.num_subcores
_NW = _NC * _NS
_BPW = _B // _NW
_CHUNK = 128
_NCHUNK = _BPW // _CHUNK
_NXFER = _L * _NCHUNK
_DEPTH = 8


def _fire(t, weight_hbm, idx_v, acc_v, sem):
    k = t % _L
    c = t // _L
    pltpu.async_copy(
        weight_hbm.at[idx_v.at[k, c]],
        acc_v.at[pl.ds(c * _CHUNK, _CHUNK)],
        sem,
        add=True,
    )


def _drain(weight_hbm, idx_v, acc_v, sem):
    pltpu.make_async_copy(
        weight_hbm.at[idx_v.at[0, 0]],
        acc_v.at[pl.ds(0, _CHUNK)],
        sem,
    ).wait()


def _emb_body(seg_hbm, weight_hbm, out_hbm, idx_v, acc_v, sem):
    wid = lax.axis_index("s") * _NC + lax.axis_index("c")

    pltpu.sync_copy(seg_hbm.at[wid], idx_v)

    zeros = jnp.zeros((16,), jnp.float32)

    def zbody(i, _):
        for j in range(_EMB // 16):
            acc_v[i, pl.ds(j * 16, 16)] = zeros
        return 0

    lax.fori_loop(0, _BPW, zbody, 0)

    for t in range(_DEPTH):
        _fire(t, weight_hbm, idx_v, acc_v, sem)

    def pbody(t, _):
        _drain(weight_hbm, idx_v, acc_v, sem)
        _fire(t + _DEPTH, weight_hbm, idx_v, acc_v, sem)
        return 0

    lax.fori_loop(0, _NXFER - _DEPTH, pbody, 0)

    for _ in range(_DEPTH):
        _drain(weight_hbm, idx_v, acc_v, sem)

    inv = jnp.float32(1.0 / _L)

    def sbody(i, _):
        for j in range(_EMB // 16):
            sl = pl.ds(j * 16, 16)
            acc_v[i, sl] = acc_v[i, sl] * inv
        return 0

    lax.fori_loop(0, _BPW, sbody, 0)

    pltpu.sync_copy(acc_v, out_hbm.at[pl.ds(wid * _BPW, _BPW)])


_emb_kernel = functools.partial(
    pl.kernel,
    out_type=jax.ShapeDtypeStruct((_B, _EMB), jnp.float32),
    mesh=plsc.VectorSubcoreMesh(core_axis_name="c", subcore_axis_name="s"),
    scratch_types=[
        pltpu.VMEM((_L, _NCHUNK, _CHUNK), jnp.int32),
        pltpu.VMEM((_BPW, _EMB), jnp.float32),
        pltpu.SemaphoreType.DMA,
    ],
    compiler_params=pltpu.CompilerParams(use_tc_tiling_on_sc=False),
)(_emb_body)


def kernel(seg_ids, weight):
    seg_r = (
        seg_ids.reshape(_NW, _BPW, _L)
        .transpose(0, 2, 1)
        .reshape(_NW, _L, _NCHUNK, _CHUNK)
    )
    return _emb_kernel(seg_r, weight)
